# baseline (speedup 1.0000x reference)
import functools

import jax
import jax.numpy as jnp
from jax import lax
from jax.experimental import pallas as pl
from jax.experimental.pallas import tpu as pltpu
from jax.experimental.pallas import tpu_sc as plsc

T = 2048
D = 1024
F = 4096
E = 8
LN = 128
CAP = 2048
ROWS = E * CAP
BF = 512
KF = F // BF
SUB = 128
NSUB = CAP // SUB
NW = 32
TPW = T // NW
CHT = 512


def _router_body(x_ref, wg_ref, r0_ref, r1_ref, w1o_ref, w2o_ref, cnt_ref,
                 loss_ref):
    x = x_ref[...]
    logits = lax.dot_general(x, wg_ref[...], (((1,), (1,)), ((), ())))
    col = lax.broadcasted_iota(jnp.int32, (T, LN), 1)
    valid = col < E
    lm = jnp.where(valid, logits, jnp.float32(-1e30))
    m = jnp.max(lm, axis=1, keepdims=True)
    ex = jnp.where(valid, jnp.exp(lm - m), 0.0)
    probs = ex / jnp.sum(ex, axis=1, keepdims=True)
    p1 = jnp.max(probs, axis=1, keepdims=True)
    i1 = jnp.min(jnp.where(probs == p1, col, jnp.int32(LN)), axis=1,
                 keepdims=True)
    probs2 = jnp.where(col == i1, -1.0, probs)
    p2 = jnp.max(probs2, axis=1, keepdims=True)
    i2 = jnp.min(jnp.where(probs2 == p2, col, jnp.int32(LN)), axis=1,
                 keepdims=True)
    den = p1 + p2 + 1e-8
    w1 = p1 / den
    w2 = p2 / den
    oh1 = (col == i1).astype(jnp.float32)
    oh2 = (col == i2).astype(jnp.float32)
    occ = oh1 + oh2
    ri = lax.broadcasted_iota(jnp.int32, (CHT, CHT), 0)
    ci = lax.broadcasted_iota(jnp.int32, (CHT, CHT), 1)
    ltri = (ri > ci).astype(jnp.float32)
    carry = jnp.zeros((1, LN), jnp.float32)
    chunks = []
    for ch in range(T // CHT):
        blk = lax.slice(occ, (ch * CHT, 0), ((ch + 1) * CHT, LN))
        cum = lax.dot_general(ltri, blk, (((1,), (0,)), ((), ())),
                              preferred_element_type=jnp.float32) + carry
        chunks.append(cum)
        carry = carry + jnp.sum(blk, axis=0, keepdims=True)
    pos = jnp.concatenate(chunks, axis=0)
    counts = carry
    vals = col.astype(jnp.float32) * jnp.float32(CAP) + pos
    r0 = jnp.sum(oh1 * vals, axis=1, keepdims=True)
    r1 = jnp.sum(oh2 * vals, axis=1, keepdims=True)
    r0_ref[...] = r0.astype(jnp.int32)
    r1_ref[...] = r1.astype(jnp.int32)
    w1o_ref[...] = jnp.broadcast_to(w1, (T, LN))
    w2o_ref[...] = jnp.broadcast_to(w2, (T, LN))
    cnt_ref[...] = counts.astype(jnp.int32)
    meanp = jnp.sum(probs, axis=0, keepdims=True) * jnp.float32(1.0 / T)
    usage = counts * jnp.float32(1.0 / (2 * T))
    loss_ref[...] = jnp.sum(meanp * usage, axis=1,
                            keepdims=True) * jnp.float32(E)


def _router_call(xf, wgp):
    return pl.pallas_call(
        _router_body,
        out_shape=(
            jax.ShapeDtypeStruct((T, 1), jnp.int32),
            jax.ShapeDtypeStruct((T, 1), jnp.int32),
            jax.ShapeDtypeStruct((T, LN), jnp.float32),
            jax.ShapeDtypeStruct((T, LN), jnp.float32),
            jax.ShapeDtypeStruct((1, LN), jnp.int32),
            jax.ShapeDtypeStruct((1, 1), jnp.float32),
        ),
    )(xf, wgp)


def _gelu(h):
    return 0.5 * h * (1.0 + lax.erf(h * 0.7071067811865476))


def _mlp_body(cnt_ref, xg_ref, wgt_ref, w1_ref, w2_ref, y_ref, xb_ref):
    e = pl.program_id(0)
    f = pl.program_id(1)
    c = cnt_ref[e]
    w1b = w1_ref[0].astype(jnp.bfloat16)
    w2b = w2_ref[0].astype(jnp.bfloat16)
    for sub in range(NSUB):
        @pl.when(c > sub * SUB)
        def _():
            sl = pl.ds(sub * SUB, SUB)

            @pl.when(f == 0)
            def _():
                xb_ref[sl, :] = xg_ref[sl, :].astype(jnp.bfloat16)

            xs = xb_ref[sl, :]
            h = lax.dot_general(xs, w1b, (((1,), (1,)), ((), ())),
                                preferred_element_type=jnp.float32)
            h = _gelu(h)
            part = lax.dot_general(h.astype(jnp.bfloat16), w2b,
                                   (((1,), (1,)), ((), ())),
                                   preferred_element_type=jnp.float32)

            @pl.when(f == 0)
            def _():
                y_ref[sl, :] = part

            @pl.when(jnp.logical_and(f > 0, f < KF - 1))
            def _():
                y_ref[sl, :] = y_ref[sl, :] + part

            @pl.when(f == KF - 1)
            def _():
                y_ref[sl, :] = (y_ref[sl, :] + part) * wgt_ref[sl, 0:1]


def _mlp_call(counts, xg, wgt, w1, w2):
    grid_spec = pltpu.PrefetchScalarGridSpec(
        num_scalar_prefetch=1,
        grid=(E, KF),
        in_specs=[
            pl.BlockSpec((CAP, D), lambda e, f, cnt: (e, 0)),
            pl.BlockSpec((CAP, LN), lambda e, f, cnt: (e, 0)),
            pl.BlockSpec((1, BF, D), lambda e, f, cnt: (e, f, 0)),
            pl.BlockSpec((1, D, BF), lambda e, f, cnt: (e, 0, f)),
        ],
        out_specs=pl.BlockSpec((CAP, D), lambda e, f, cnt: (e, 0)),
        scratch_shapes=[pltpu.VMEM((CAP, D), jnp.bfloat16)],
    )
    return pl.pallas_call(
        _mlp_body,
        grid_spec=grid_spec,
        out_shape=jax.ShapeDtypeStruct((ROWS, D), jnp.float32),
        compiler_params=pltpu.CompilerParams(
            dimension_semantics=("arbitrary", "arbitrary")),
    )(counts, xg, wgt, w1, w2)


def _sc_dispatch_body(x_hbm, r0_hbm, r1_hbm, wa_hbm, wb_hbm, xg_hbm, wgt_hbm,
                      xbuf, wbuf, i0, i1, sem):
    wid = lax.axis_index("s") * 2 + lax.axis_index("c")
    base = wid * TPW
    pltpu.sync_copy(r0_hbm.at[wid], i0)
    pltpu.sync_copy(r1_hbm.at[wid], i1)
    pltpu.sync_copy(x_hbm.at[pl.ds(base, TPW)], xbuf)
    pltpu.async_copy(xbuf, xg_hbm.at[i0], sem).wait()
    pltpu.async_copy(xbuf, xg_hbm.at[i1], sem).wait()
    pltpu.sync_copy(wa_hbm.at[pl.ds(base, TPW)], wbuf)
    pltpu.async_copy(wbuf, wgt_hbm.at[i0], sem).wait()
    pltpu.sync_copy(wb_hbm.at[pl.ds(base, TPW)], wbuf)
    pltpu.async_copy(wbuf, wgt_hbm.at[i1], sem).wait()


def _sc_dispatch(xf, r0m, r1m, w1b, w2b):
    mesh = plsc.VectorSubcoreMesh(core_axis_name="c", subcore_axis_name="s")
    fn = functools.partial(
        pl.kernel,
        mesh=mesh,
        out_type=(
            jax.ShapeDtypeStruct((ROWS, D), jnp.float32),
            jax.ShapeDtypeStruct((ROWS, LN), jnp.float32),
        ),
        scratch_types=[
            pltpu.VMEM((TPW, D), jnp.float32),
            pltpu.VMEM((TPW, LN), jnp.float32),
            pltpu.VMEM((TPW,), jnp.int32),
            pltpu.VMEM((TPW,), jnp.int32),
            pltpu.SemaphoreType.DMA,
        ],
    )(_sc_dispatch_body)
    return fn(xf, r0m, r1m, w1b, w2b)


def _sc_combine_body(y_hbm, r0_hbm, r1_hbm, o_hbm, b0, b1, i0, i1, sem):
    wid = lax.axis_index("s") * 2 + lax.axis_index("c")
    base = wid * TPW
    pltpu.sync_copy(r0_hbm.at[wid], i0)
    pltpu.sync_copy(r1_hbm.at[wid], i1)
    for hh in range(2):
        cp0 = pltpu.async_copy(y_hbm.at[i0.at[hh]], b0, sem)
        cp1 = pltpu.async_copy(y_hbm.at[i1.at[hh]], b1, sem)
        cp0.wait()
        cp1.wait()

        def body(r, _):
            for k in range(D // 16):
                ksl = pl.ds(k * 16, 16)
                b0[r, ksl] = b0[r, ksl] + b1[r, ksl]
            return 0

        lax.fori_loop(0, TPW // 2, body, 0)
        pltpu.sync_copy(b0, o_hbm.at[pl.ds(base + hh * (TPW // 2), TPW // 2)])


def _sc_combine(y, r0c, r1c):
    mesh = plsc.VectorSubcoreMesh(core_axis_name="c", subcore_axis_name="s")
    fn = functools.partial(
        pl.kernel,
        mesh=mesh,
        out_type=jax.ShapeDtypeStruct((T, D), jnp.float32),
        scratch_types=[
            pltpu.VMEM((TPW // 2, D), jnp.float32),
            pltpu.VMEM((TPW // 2, D), jnp.float32),
            pltpu.VMEM((2, TPW // 2), jnp.int32),
            pltpu.VMEM((2, TPW // 2), jnp.int32),
            pltpu.SemaphoreType.DMA,
        ],
    )(_sc_combine_body)
    return fn(y, r0c, r1c)


def kernel(x, Wg, W1, W2):
    B, S, _ = x.shape
    xf = x.reshape(T, D)
    wgp = jnp.pad(Wg, ((0, LN - E), (0, 0)))
    r0, r1, w1b, w2b, counts, loss = _router_call(xf, wgp)
    r0f = r0.reshape(T)
    r1f = r1.reshape(T)
    xg, wgt = _sc_dispatch(xf, r0f.reshape(NW, TPW), r1f.reshape(NW, TPW),
                           w1b, w2b)
    cnt8 = counts[0, :E]
    y = _mlp_call(cnt8, xg, wgt, W1, W2)
    out = _sc_combine(y, r0f.reshape(NW, 2, TPW // 2),
                      r1f.reshape(NW, 2, TPW // 2))
    return out.reshape(B, S, D), loss[0, 0]


# speedup vs baseline: 1.3333x; 1.3333x over previous
import functools

import jax
import jax.numpy as jnp
from jax import lax
from jax.experimental import pallas as pl
from jax.experimental.pallas import tpu as pltpu
from jax.experimental.pallas import tpu_sc as plsc

T = 2048
D = 1024
F = 4096
E = 8
LN = 128
CAP = 2048
ROWS = E * CAP
BF = 512
KF = F // BF
SUB = 256
NSUB = CAP // SUB
NW = 32
TPW = T // NW
CHT = 512


def _router_body(x_ref, wg_ref, r0_ref, r1_ref, w1o_ref, w2o_ref, cnt_ref,
                 loss_ref):
    x = x_ref[...]
    logits = lax.dot_general(x, wg_ref[...], (((1,), (1,)), ((), ())))
    col = lax.broadcasted_iota(jnp.int32, (T, LN), 1)
    valid = col < E
    lm = jnp.where(valid, logits, jnp.float32(-1e30))
    m = jnp.max(lm, axis=1, keepdims=True)
    ex = jnp.where(valid, jnp.exp(lm - m), 0.0)
    probs = ex / jnp.sum(ex, axis=1, keepdims=True)
    p1 = jnp.max(probs, axis=1, keepdims=True)
    i1 = jnp.min(jnp.where(probs == p1, col, jnp.int32(LN)), axis=1,
                 keepdims=True)
    probs2 = jnp.where(col == i1, -1.0, probs)
    p2 = jnp.max(probs2, axis=1, keepdims=True)
    i2 = jnp.min(jnp.where(probs2 == p2, col, jnp.int32(LN)), axis=1,
                 keepdims=True)
    den = p1 + p2 + 1e-8
    w1 = p1 / den
    w2 = p2 / den
    oh1 = (col == i1).astype(jnp.float32)
    oh2 = (col == i2).astype(jnp.float32)
    occ = oh1 + oh2
    ri = lax.broadcasted_iota(jnp.int32, (CHT, CHT), 0)
    ci = lax.broadcasted_iota(jnp.int32, (CHT, CHT), 1)
    ltri = (ri > ci).astype(jnp.float32)
    carry = jnp.zeros((1, LN), jnp.float32)
    chunks = []
    for ch in range(T // CHT):
        blk = lax.slice(occ, (ch * CHT, 0), ((ch + 1) * CHT, LN))
        cum = lax.dot_general(ltri, blk, (((1,), (0,)), ((), ())),
                              preferred_element_type=jnp.float32) + carry
        chunks.append(cum)
        carry = carry + jnp.sum(blk, axis=0, keepdims=True)
    pos = jnp.concatenate(chunks, axis=0)
    counts = carry
    vals = col.astype(jnp.float32) * jnp.float32(CAP) + pos
    r0 = jnp.sum(oh1 * vals, axis=1, keepdims=True)
    r1 = jnp.sum(oh2 * vals, axis=1, keepdims=True)
    r0_ref[...] = r0.astype(jnp.int32)
    r1_ref[...] = r1.astype(jnp.int32)
    w1o_ref[...] = jnp.broadcast_to(w1, (T, LN))
    w2o_ref[...] = jnp.broadcast_to(w2, (T, LN))
    cnt_ref[...] = counts.astype(jnp.int32)
    meanp = jnp.sum(probs, axis=0, keepdims=True) * jnp.float32(1.0 / T)
    usage = counts * jnp.float32(1.0 / (2 * T))
    loss_ref[...] = jnp.sum(meanp * usage, axis=1,
                            keepdims=True) * jnp.float32(E)


def _router_call(xf, wgp):
    return pl.pallas_call(
        _router_body,
        out_shape=(
            jax.ShapeDtypeStruct((T, 1), jnp.int32),
            jax.ShapeDtypeStruct((T, 1), jnp.int32),
            jax.ShapeDtypeStruct((T, LN), jnp.float32),
            jax.ShapeDtypeStruct((T, LN), jnp.float32),
            jax.ShapeDtypeStruct((1, LN), jnp.int32),
            jax.ShapeDtypeStruct((1, 1), jnp.float32),
        ),
    )(xf, wgp)


def _gelu(h):
    return 0.5 * h * (1.0 + lax.erf(h * 0.7071067811865476))


def _mlp_body(cnt_ref, xg_ref, wgt_ref, w1_ref, w2_ref, y_ref, xb_ref):
    e = pl.program_id(0)
    f = pl.program_id(1)
    c = cnt_ref[e]
    w1b = w1_ref[0].astype(jnp.bfloat16)
    w2b = w2_ref[0].astype(jnp.bfloat16)
    for sub in range(NSUB):
        @pl.when(c > sub * SUB)
        def _():
            sl = pl.ds(sub * SUB, SUB)

            @pl.when(f == 0)
            def _():
                xb_ref[sl, :] = xg_ref[sl, :].astype(jnp.bfloat16)

            xs = xb_ref[sl, :]
            h = lax.dot_general(xs, w1b, (((1,), (1,)), ((), ())),
                                preferred_element_type=jnp.float32)
            h = _gelu(h)
            part = lax.dot_general(h.astype(jnp.bfloat16), w2b,
                                   (((1,), (1,)), ((), ())),
                                   preferred_element_type=jnp.float32)

            @pl.when(f == 0)
            def _():
                y_ref[sl, :] = part

            @pl.when(jnp.logical_and(f > 0, f < KF - 1))
            def _():
                y_ref[sl, :] = y_ref[sl, :] + part

            @pl.when(f == KF - 1)
            def _():
                y_ref[sl, :] = (y_ref[sl, :] + part) * wgt_ref[sl, 0:1]


def _mlp_call(counts, xg, wgt, w1, w2):
    grid_spec = pltpu.PrefetchScalarGridSpec(
        num_scalar_prefetch=1,
        grid=(E, KF),
        in_specs=[
            pl.BlockSpec((CAP, D), lambda e, f, cnt: (e, 0)),
            pl.BlockSpec((CAP, LN), lambda e, f, cnt: (e, 0)),
            pl.BlockSpec((1, BF, D), lambda e, f, cnt: (e, f, 0)),
            pl.BlockSpec((1, D, BF), lambda e, f, cnt: (e, 0, f)),
        ],
        out_specs=pl.BlockSpec((CAP, D), lambda e, f, cnt: (e, 0)),
        scratch_shapes=[pltpu.VMEM((CAP, D), jnp.bfloat16)],
    )
    return pl.pallas_call(
        _mlp_body,
        grid_spec=grid_spec,
        out_shape=jax.ShapeDtypeStruct((ROWS, D), jnp.float32),
        compiler_params=pltpu.CompilerParams(
            dimension_semantics=("arbitrary", "arbitrary")),
    )(counts, xg, wgt, w1, w2)


def _sc_dispatch_body(x_hbm, r0_hbm, r1_hbm, wa_hbm, wb_hbm, xg_hbm, wgt_hbm,
                      xbuf, wbuf, i0, i1, sem):
    wid = lax.axis_index("s") * 2 + lax.axis_index("c")
    base = wid * TPW
    pltpu.sync_copy(r0_hbm.at[wid], i0)
    pltpu.sync_copy(r1_hbm.at[wid], i1)
    pltpu.sync_copy(x_hbm.at[pl.ds(base, TPW)], xbuf)
    pltpu.async_copy(xbuf, xg_hbm.at[i0], sem).wait()
    pltpu.async_copy(xbuf, xg_hbm.at[i1], sem).wait()
    pltpu.sync_copy(wa_hbm.at[pl.ds(base, TPW)], wbuf)
    pltpu.async_copy(wbuf, wgt_hbm.at[i0], sem).wait()
    pltpu.sync_copy(wb_hbm.at[pl.ds(base, TPW)], wbuf)
    pltpu.async_copy(wbuf, wgt_hbm.at[i1], sem).wait()


def _sc_dispatch(xf, r0m, r1m, w1b, w2b):
    mesh = plsc.VectorSubcoreMesh(core_axis_name="c", subcore_axis_name="s")
    fn = functools.partial(
        pl.kernel,
        mesh=mesh,
        out_type=(
            jax.ShapeDtypeStruct((ROWS, D), jnp.float32),
            jax.ShapeDtypeStruct((ROWS, LN), jnp.float32),
        ),
        scratch_types=[
            pltpu.VMEM((TPW, D), jnp.float32),
            pltpu.VMEM((TPW, LN), jnp.float32),
            pltpu.VMEM((TPW,), jnp.int32),
            pltpu.VMEM((TPW,), jnp.int32),
            pltpu.SemaphoreType.DMA,
        ],
    )(_sc_dispatch_body)
    return fn(xf, r0m, r1m, w1b, w2b)


def _sc_combine_body(y_hbm, r0_hbm, r1_hbm, o_hbm, b0, b1, i0, i1, sem):
    wid = lax.axis_index("s") * 2 + lax.axis_index("c")
    base = wid * TPW
    pltpu.sync_copy(r0_hbm.at[wid], i0)
    pltpu.sync_copy(r1_hbm.at[wid], i1)
    for hh in range(2):
        cp0 = pltpu.async_copy(y_hbm.at[i0.at[hh]], b0, sem)
        cp1 = pltpu.async_copy(y_hbm.at[i1.at[hh]], b1, sem)
        cp0.wait()
        cp1.wait()

        def body(r, _):
            for k in range(D // 16):
                ksl = pl.ds(k * 16, 16)
                b0[r, ksl] = b0[r, ksl] + b1[r, ksl]
            return 0

        lax.fori_loop(0, TPW // 2, body, 0)
        pltpu.sync_copy(b0, o_hbm.at[pl.ds(base + hh * (TPW // 2), TPW // 2)])


def _sc_combine(y, r0c, r1c):
    mesh = plsc.VectorSubcoreMesh(core_axis_name="c", subcore_axis_name="s")
    fn = functools.partial(
        pl.kernel,
        mesh=mesh,
        out_type=jax.ShapeDtypeStruct((T, D), jnp.float32),
        scratch_types=[
            pltpu.VMEM((TPW // 2, D), jnp.float32),
            pltpu.VMEM((TPW // 2, D), jnp.float32),
            pltpu.VMEM((2, TPW // 2), jnp.int32),
            pltpu.VMEM((2, TPW // 2), jnp.int32),
            pltpu.SemaphoreType.DMA,
        ],
    )(_sc_combine_body)
    return fn(y, r0c, r1c)


def kernel(x, Wg, W1, W2):
    B, S, _ = x.shape
    xf = x.reshape(T, D)
    wgp = jnp.pad(Wg, ((0, LN - E), (0, 0)))
    r0, r1, w1b, w2b, counts, loss = _router_call(xf, wgp)
    r0f = r0.reshape(T)
    r1f = r1.reshape(T)
    xg, wgt = _sc_dispatch(xf, r0f.reshape(NW, TPW), r1f.reshape(NW, TPW),
                           w1b, w2b)
    cnt8 = counts[0, :E]
    y = _mlp_call(cnt8, xg, wgt, W1, W2)
    out = _sc_combine(y, r0f.reshape(NW, 2, TPW // 2),
                      r1f.reshape(NW, 2, TPW // 2))
    return out.reshape(B, S, D), loss[0, 0]


# speedup vs baseline: 1.6415x; 1.2311x over previous
import functools

import jax
import jax.numpy as jnp
from jax import lax
from jax.experimental import pallas as pl
from jax.experimental.pallas import tpu as pltpu
from jax.experimental.pallas import tpu_sc as plsc

T = 2048
D = 1024
F = 4096
E = 8
LN = 128
CAP = 2048
ROWS = E * CAP
BF = 1024
KF = F // BF
SUB = 256
NSUB = CAP // SUB
NW = 32
TPW = T // NW
CHT = 512


def _router_body(x_ref, wg_ref, r0_ref, r1_ref, w1o_ref, w2o_ref, cnt_ref,
                 loss_ref):
    x = x_ref[...]
    logits = lax.dot_general(x, wg_ref[...], (((1,), (1,)), ((), ())))
    col = lax.broadcasted_iota(jnp.int32, (T, LN), 1)
    valid = col < E
    lm = jnp.where(valid, logits, jnp.float32(-1e30))
    m = jnp.max(lm, axis=1, keepdims=True)
    ex = jnp.where(valid, jnp.exp(lm - m), 0.0)
    probs = ex / jnp.sum(ex, axis=1, keepdims=True)
    p1 = jnp.max(probs, axis=1, keepdims=True)
    i1 = jnp.min(jnp.where(probs == p1, col, jnp.int32(LN)), axis=1,
                 keepdims=True)
    probs2 = jnp.where(col == i1, -1.0, probs)
    p2 = jnp.max(probs2, axis=1, keepdims=True)
    i2 = jnp.min(jnp.where(probs2 == p2, col, jnp.int32(LN)), axis=1,
                 keepdims=True)
    den = p1 + p2 + 1e-8
    w1 = p1 / den
    w2 = p2 / den
    oh1 = (col == i1).astype(jnp.float32)
    oh2 = (col == i2).astype(jnp.float32)
    occ = oh1 + oh2
    ri = lax.broadcasted_iota(jnp.int32, (CHT, CHT), 0)
    ci = lax.broadcasted_iota(jnp.int32, (CHT, CHT), 1)
    ltri = (ri > ci).astype(jnp.float32)
    carry = jnp.zeros((1, LN), jnp.float32)
    chunks = []
    for ch in range(T // CHT):
        blk = lax.slice(occ, (ch * CHT, 0), ((ch + 1) * CHT, LN))
        cum = lax.dot_general(ltri, blk, (((1,), (0,)), ((), ())),
                              preferred_element_type=jnp.float32) + carry
        chunks.append(cum)
        carry = carry + jnp.sum(blk, axis=0, keepdims=True)
    pos = jnp.concatenate(chunks, axis=0)
    counts = carry
    vals = col.astype(jnp.float32) * jnp.float32(CAP) + pos
    r0 = jnp.sum(oh1 * vals, axis=1, keepdims=True)
    r1 = jnp.sum(oh2 * vals, axis=1, keepdims=True)
    r0_ref[...] = r0.astype(jnp.int32)
    r1_ref[...] = r1.astype(jnp.int32)
    w1o_ref[...] = jnp.broadcast_to(w1, (T, LN))
    w2o_ref[...] = jnp.broadcast_to(w2, (T, LN))
    cnt_ref[...] = counts.astype(jnp.int32)
    meanp = jnp.sum(probs, axis=0, keepdims=True) * jnp.float32(1.0 / T)
    usage = counts * jnp.float32(1.0 / (2 * T))
    loss_ref[...] = jnp.sum(meanp * usage, axis=1,
                            keepdims=True) * jnp.float32(E)


def _router_call(xf, wgp):
    return pl.pallas_call(
        _router_body,
        out_shape=(
            jax.ShapeDtypeStruct((T, 1), jnp.int32),
            jax.ShapeDtypeStruct((T, 1), jnp.int32),
            jax.ShapeDtypeStruct((T, LN), jnp.float32),
            jax.ShapeDtypeStruct((T, LN), jnp.float32),
            jax.ShapeDtypeStruct((1, LN), jnp.int32),
            jax.ShapeDtypeStruct((1, 1), jnp.float32),
        ),
    )(xf, wgp)


def _gelu(h):
    return 0.5 * h * (1.0 + lax.erf(h * 0.7071067811865476))


def _mlp_body(cnt_ref, xg_ref, wgt_ref, w1_ref, w2_ref, y_ref):
    e = pl.program_id(0)
    f = pl.program_id(1)
    c = cnt_ref[e]
    w1b = w1_ref[0].astype(jnp.bfloat16)
    w2b = w2_ref[0].astype(jnp.bfloat16)
    for sub in range(NSUB):
        @pl.when(c > sub * SUB)
        def _():
            sl = pl.ds(sub * SUB, SUB)
            xs = xg_ref[sl, :].astype(jnp.bfloat16)
            h = lax.dot_general(xs, w1b, (((1,), (1,)), ((), ())),
                                preferred_element_type=jnp.float32)
            h = _gelu(h)
            part = lax.dot_general(h.astype(jnp.bfloat16), w2b,
                                   (((1,), (1,)), ((), ())),
                                   preferred_element_type=jnp.float32)

            @pl.when(f == 0)
            def _():
                y_ref[sl, :] = part

            @pl.when(jnp.logical_and(f > 0, f < KF - 1))
            def _():
                y_ref[sl, :] = y_ref[sl, :] + part

            @pl.when(f == KF - 1)
            def _():
                y_ref[sl, :] = (y_ref[sl, :] + part) * wgt_ref[sl, 0:1]


def _mlp_call(counts, xg, wgt, w1, w2):
    grid_spec = pltpu.PrefetchScalarGridSpec(
        num_scalar_prefetch=1,
        grid=(E, KF),
        in_specs=[
            pl.BlockSpec((CAP, D), lambda e, f, cnt: (e, 0)),
            pl.BlockSpec((CAP, LN), lambda e, f, cnt: (e, 0)),
            pl.BlockSpec((1, BF, D), lambda e, f, cnt: (e, f, 0)),
            pl.BlockSpec((1, D, BF), lambda e, f, cnt: (e, 0, f)),
        ],
        out_specs=pl.BlockSpec((CAP, D), lambda e, f, cnt: (e, 0)),
    )
    return pl.pallas_call(
        _mlp_body,
        grid_spec=grid_spec,
        out_shape=jax.ShapeDtypeStruct((ROWS, D), jnp.float32),
        compiler_params=pltpu.CompilerParams(
            dimension_semantics=("arbitrary", "arbitrary")),
    )(counts, xg, wgt, w1, w2)


def _sc_dispatch_body(x_hbm, r0_hbm, r1_hbm, wa_hbm, wb_hbm, xg_hbm, wgt_hbm,
                      xbuf, wbuf, i0, i1, sem):
    wid = lax.axis_index("s") * 2 + lax.axis_index("c")
    base = wid * TPW
    pltpu.sync_copy(r0_hbm.at[wid], i0)
    pltpu.sync_copy(r1_hbm.at[wid], i1)
    pltpu.sync_copy(x_hbm.at[pl.ds(base, TPW)], xbuf)
    pltpu.async_copy(xbuf, xg_hbm.at[i0], sem).wait()
    pltpu.async_copy(xbuf, xg_hbm.at[i1], sem).wait()
    pltpu.sync_copy(wa_hbm.at[pl.ds(base, TPW)], wbuf)
    pltpu.async_copy(wbuf, wgt_hbm.at[i0], sem).wait()
    pltpu.sync_copy(wb_hbm.at[pl.ds(base, TPW)], wbuf)
    pltpu.async_copy(wbuf, wgt_hbm.at[i1], sem).wait()


def _sc_dispatch(xf, r0m, r1m, w1b, w2b):
    mesh = plsc.VectorSubcoreMesh(core_axis_name="c", subcore_axis_name="s")
    fn = functools.partial(
        pl.kernel,
        mesh=mesh,
        out_type=(
            jax.ShapeDtypeStruct((ROWS, D), jnp.float32),
            jax.ShapeDtypeStruct((ROWS, LN), jnp.float32),
        ),
        scratch_types=[
            pltpu.VMEM((TPW, D), jnp.float32),
            pltpu.VMEM((TPW, LN), jnp.float32),
            pltpu.VMEM((TPW,), jnp.int32),
            pltpu.VMEM((TPW,), jnp.int32),
            pltpu.SemaphoreType.DMA,
        ],
    )(_sc_dispatch_body)
    return fn(xf, r0m, r1m, w1b, w2b)


def _sc_combine_body(y_hbm, r0_hbm, r1_hbm, o_hbm, b0, b1, i0, i1, sem):
    wid = lax.axis_index("s") * 2 + lax.axis_index("c")
    base = wid * TPW
    pltpu.sync_copy(r0_hbm.at[wid], i0)
    pltpu.sync_copy(r1_hbm.at[wid], i1)
    for hh in range(2):
        cp0 = pltpu.async_copy(y_hbm.at[i0.at[hh]], b0, sem)
        cp1 = pltpu.async_copy(y_hbm.at[i1.at[hh]], b1, sem)
        cp0.wait()
        cp1.wait()

        def body(r, _):
            for k in range(D // 16):
                ksl = pl.ds(k * 16, 16)
                b0[r, ksl] = b0[r, ksl] + b1[r, ksl]
            return 0

        lax.fori_loop(0, TPW // 2, body, 0)
        pltpu.sync_copy(b0, o_hbm.at[pl.ds(base + hh * (TPW // 2), TPW // 2)])


def _sc_combine(y, r0c, r1c):
    mesh = plsc.VectorSubcoreMesh(core_axis_name="c", subcore_axis_name="s")
    fn = functools.partial(
        pl.kernel,
        mesh=mesh,
        out_type=jax.ShapeDtypeStruct((T, D), jnp.float32),
        scratch_types=[
            pltpu.VMEM((TPW // 2, D), jnp.float32),
            pltpu.VMEM((TPW // 2, D), jnp.float32),
            pltpu.VMEM((2, TPW // 2), jnp.int32),
            pltpu.VMEM((2, TPW // 2), jnp.int32),
            pltpu.SemaphoreType.DMA,
        ],
    )(_sc_combine_body)
    return fn(y, r0c, r1c)


def kernel(x, Wg, W1, W2):
    B, S, _ = x.shape
    xf = x.reshape(T, D)
    wgp = jnp.pad(Wg, ((0, LN - E), (0, 0)))
    r0, r1, w1b, w2b, counts, loss = _router_call(xf, wgp)
    r0f = r0.reshape(T)
    r1f = r1.reshape(T)
    xg, wgt = _sc_dispatch(xf, r0f.reshape(NW, TPW), r1f.reshape(NW, TPW),
                           w1b, w2b)
    cnt8 = counts[0, :E]
    y = _mlp_call(cnt8, xg, wgt, W1, W2)
    out = _sc_combine(y, r0f.reshape(NW, 2, TPW // 2),
                      r1f.reshape(NW, 2, TPW // 2))
    return out.reshape(B, S, D), loss[0, 0]


# speedup vs baseline: 1.6521x; 1.0064x over previous
import functools

import jax
import jax.numpy as jnp
from jax import lax
from jax.experimental import pallas as pl
from jax.experimental.pallas import tpu as pltpu
from jax.experimental.pallas import tpu_sc as plsc

T = 2048
D = 1024
F = 4096
E = 8
LN = 128
CAP = 2048
ROWS = E * CAP
BF = 1024
KF = F // BF
HC = 256
SUB = 256
NSUB = CAP // SUB
NW = 32
TPW = T // NW
CR = 16
CQ = TPW // CR
CHT = 512


def _router_body(x_ref, wg_ref, r0_ref, r1_ref, w1o_ref, w2o_ref, cnt_ref,
                 loss_ref):
    x = x_ref[...]
    logits = lax.dot_general(x, wg_ref[...], (((1,), (1,)), ((), ())))
    col = lax.broadcasted_iota(jnp.int32, (T, E), 1)
    m = jnp.max(logits, axis=1, keepdims=True)
    ex = jnp.exp(logits - m)
    probs = ex / jnp.sum(ex, axis=1, keepdims=True)
    p1 = jnp.max(probs, axis=1, keepdims=True)
    i1 = jnp.min(jnp.where(probs == p1, col, jnp.int32(E)), axis=1,
                 keepdims=True)
    probs2 = jnp.where(col == i1, -1.0, probs)
    p2 = jnp.max(probs2, axis=1, keepdims=True)
    i2 = jnp.min(jnp.where(probs2 == p2, col, jnp.int32(E)), axis=1,
                 keepdims=True)
    den = p1 + p2 + 1e-8
    w1 = p1 / den
    w2 = p2 / den
    oh1 = (col == i1).astype(jnp.float32)
    oh2 = (col == i2).astype(jnp.float32)
    occ = oh1 + oh2
    ri = lax.broadcasted_iota(jnp.int32, (CHT, CHT), 0)
    ci = lax.broadcasted_iota(jnp.int32, (CHT, CHT), 1)
    ltri = (ri > ci).astype(jnp.float32)
    carry = jnp.zeros((1, E), jnp.float32)
    chunks = []
    for ch in range(T // CHT):
        blk = lax.slice(occ, (ch * CHT, 0), ((ch + 1) * CHT, E))
        cum = lax.dot_general(ltri, blk, (((1,), (0,)), ((), ())),
                              preferred_element_type=jnp.float32) + carry
        chunks.append(cum)
        carry = carry + jnp.sum(blk, axis=0, keepdims=True)
    pos = jnp.concatenate(chunks, axis=0)
    counts = carry
    vals = col.astype(jnp.float32) * jnp.float32(CAP) + pos
    r0 = jnp.sum(oh1 * vals, axis=1, keepdims=True)
    r1 = jnp.sum(oh2 * vals, axis=1, keepdims=True)
    r0_ref[...] = r0.astype(jnp.int32)
    r1_ref[...] = r1.astype(jnp.int32)
    w1o_ref[...] = jnp.broadcast_to(w1, (T, LN))
    w2o_ref[...] = jnp.broadcast_to(w2, (T, LN))
    cnt_ref[...] = counts.astype(jnp.int32)
    meanp = jnp.sum(probs, axis=0, keepdims=True) * jnp.float32(1.0 / T)
    usage = counts * jnp.float32(1.0 / (2 * T))
    loss_ref[...] = jnp.sum(meanp * usage, axis=1,
                            keepdims=True) * jnp.float32(E)


def _router_call(xf, wg):
    return pl.pallas_call(
        _router_body,
        out_shape=(
            jax.ShapeDtypeStruct((T, 1), jnp.int32),
            jax.ShapeDtypeStruct((T, 1), jnp.int32),
            jax.ShapeDtypeStruct((T, LN), jnp.float32),
            jax.ShapeDtypeStruct((T, LN), jnp.float32),
            jax.ShapeDtypeStruct((1, E), jnp.int32),
            jax.ShapeDtypeStruct((1, 1), jnp.float32),
        ),
    )(xf, wg)


def _gelu(h):
    return 0.5 * h * (1.0 + lax.erf(h * 0.7071067811865476))


def _mlp_body(cnt_ref, xg_ref, wgt_ref, w1_ref, w2_ref, y_ref):
    e = pl.program_id(0)
    f = pl.program_id(1)
    c = cnt_ref[0, e]
    w1b = w1_ref[0].astype(jnp.bfloat16)
    w2b = w2_ref[0].astype(jnp.bfloat16)
    for sub in range(NSUB):
        @pl.when(c > sub * SUB)
        def _():
            sl = pl.ds(sub * SUB, SUB)
            xs = xg_ref[sl, :].astype(jnp.bfloat16)
            hbs = []
            for hc in range(BF // HC):
                w1c = lax.slice(w1b, (hc * HC, 0), ((hc + 1) * HC, D))
                hcv = lax.dot_general(xs, w1c, (((1,), (1,)), ((), ())),
                                      preferred_element_type=jnp.float32)
                hbs.append(_gelu(hcv).astype(jnp.bfloat16))
            hb = jnp.concatenate(hbs, axis=1)
            part = lax.dot_general(hb, w2b, (((1,), (1,)), ((), ())),
                                   preferred_element_type=jnp.float32)

            @pl.when(f == 0)
            def _():
                y_ref[sl, :] = part

            @pl.when(jnp.logical_and(f > 0, f < KF - 1))
            def _():
                y_ref[sl, :] = y_ref[sl, :] + part

            @pl.when(f == KF - 1)
            def _():
                y_ref[sl, :] = (y_ref[sl, :] + part) * wgt_ref[sl, 0:1]


def _mlp_call(counts, xg, wgt, w1, w2):
    grid_spec = pltpu.PrefetchScalarGridSpec(
        num_scalar_prefetch=1,
        grid=(E, KF),
        in_specs=[
            pl.BlockSpec((CAP, D), lambda e, f, cnt: (e, 0)),
            pl.BlockSpec((CAP, LN), lambda e, f, cnt: (e, 0)),
            pl.BlockSpec((1, BF, D), lambda e, f, cnt: (e, f, 0)),
            pl.BlockSpec((1, D, BF), lambda e, f, cnt: (e, 0, f)),
        ],
        out_specs=pl.BlockSpec((CAP, D), lambda e, f, cnt: (e, 0)),
    )
    return pl.pallas_call(
        _mlp_body,
        grid_spec=grid_spec,
        out_shape=jax.ShapeDtypeStruct((ROWS, D), jnp.float32),
        compiler_params=pltpu.CompilerParams(
            dimension_semantics=("arbitrary", "arbitrary")),
    )(counts, xg, wgt, w1, w2)


def _sc_dispatch_body(x_hbm, r0_hbm, r1_hbm, wa_hbm, wb_hbm, xg_hbm, wgt_hbm,
                      xbuf, wabuf, wbbuf, i0, i1, sem):
    wid = lax.axis_index("s") * 2 + lax.axis_index("c")
    base = wid * TPW
    pltpu.sync_copy(r0_hbm.at[pl.ds(base, TPW)], i0)
    pltpu.sync_copy(r1_hbm.at[pl.ds(base, TPW)], i1)
    pltpu.sync_copy(x_hbm.at[pl.ds(base, TPW)], xbuf)
    pltpu.sync_copy(wa_hbm.at[pl.ds(base, TPW)], wabuf)
    pltpu.sync_copy(wb_hbm.at[pl.ds(base, TPW)], wbbuf)
    c1 = pltpu.async_copy(xbuf, xg_hbm.at[i0], sem)
    c2 = pltpu.async_copy(xbuf, xg_hbm.at[i1], sem)
    c3 = pltpu.async_copy(wabuf, wgt_hbm.at[i0], sem)
    c4 = pltpu.async_copy(wbbuf, wgt_hbm.at[i1], sem)
    c1.wait()
    c2.wait()
    c3.wait()
    c4.wait()


def _sc_dispatch(xf, r0f, r1f, w1b, w2b):
    mesh = plsc.VectorSubcoreMesh(core_axis_name="c", subcore_axis_name="s")
    fn = functools.partial(
        pl.kernel,
        mesh=mesh,
        out_type=(
            jax.ShapeDtypeStruct((ROWS, D), jnp.float32),
            jax.ShapeDtypeStruct((ROWS, LN), jnp.float32),
        ),
        scratch_types=[
            pltpu.VMEM((TPW, D), jnp.float32),
            pltpu.VMEM((TPW, LN), jnp.float32),
            pltpu.VMEM((TPW, LN), jnp.float32),
            pltpu.VMEM((TPW,), jnp.int32),
            pltpu.VMEM((TPW,), jnp.int32),
            pltpu.SemaphoreType.DMA,
        ],
    )(_sc_dispatch_body)
    return fn(xf, r0f, r1f, w1b, w2b)


def _sc_combine_body(y_hbm, r0_hbm, r1_hbm, o_hbm, b0, b1, i0, i1, sem0,
                     sem1):
    wid = lax.axis_index("s") * 2 + lax.axis_index("c")
    base = wid * TPW
    pltpu.sync_copy(r0_hbm.at[pl.ds(base, TPW)], i0)
    pltpu.sync_copy(r1_hbm.at[pl.ds(base, TPW)], i1)
    cps = [None] * CQ

    def _fire(q):
        csl = pl.ds(q * CR, CR)
        rb = q % 2
        return (pltpu.async_copy(y_hbm.at[i0.at[csl]], b0.at[rb], sem0),
                pltpu.async_copy(y_hbm.at[i1.at[csl]], b1.at[rb], sem1))

    cps[0] = _fire(0)
    cps[1] = _fire(1)
    for q in range(CQ):
        c0, c1 = cps[q]
        c0.wait()
        c1.wait()
        rb = q % 2

        def body(r, _):
            for k in range(D // 16):
                ksl = pl.ds(k * 16, 16)
                b0[rb, r, ksl] = b0[rb, r, ksl] + b1[rb, r, ksl]
            return 0

        lax.fori_loop(0, CR, body, 0)
        pltpu.sync_copy(b0.at[rb], o_hbm.at[pl.ds(base + q * CR, CR)])
        if q + 2 < CQ:
            cps[q + 2] = _fire(q + 2)


def _sc_combine(y, r0f, r1f):
    mesh = plsc.VectorSubcoreMesh(core_axis_name="c", subcore_axis_name="s")
    fn = functools.partial(
        pl.kernel,
        mesh=mesh,
        out_type=jax.ShapeDtypeStruct((T, D), jnp.float32),
        scratch_types=[
            pltpu.VMEM((2, CR, D), jnp.float32),
            pltpu.VMEM((2, CR, D), jnp.float32),
            pltpu.VMEM((TPW,), jnp.int32),
            pltpu.VMEM((TPW,), jnp.int32),
            pltpu.SemaphoreType.DMA,
            pltpu.SemaphoreType.DMA,
        ],
    )(_sc_combine_body)
    return fn(y, r0f, r1f)


def kernel(x, Wg, W1, W2):
    B, S, _ = x.shape
    xf = x.reshape(T, D)
    r0, r1, w1b, w2b, counts, loss = _router_call(xf, Wg)
    r0f = r0.reshape(T)
    r1f = r1.reshape(T)
    xg, wgt = _sc_dispatch(xf, r0f, r1f, w1b, w2b)
    y = _mlp_call(counts, xg, wgt, W1, W2)
    out = _sc_combine(y, r0f, r1f)
    return out.reshape(B, S, D), loss[0, 0]


# speedup vs baseline: 1.8674x; 1.1304x over previous
import functools

import jax
import jax.numpy as jnp
from jax import lax
from jax.experimental import pallas as pl
from jax.experimental.pallas import tpu as pltpu
from jax.experimental.pallas import tpu_sc as plsc

T = 2048
D = 1024
F = 4096
E = 8
LN = 128
CAP = 2048
CAPF = 768
BF = 1024
KF = F // BF
HC = 256
SUB = 256
NW = 32
TPW = T // NW
CR = 16
CQ = TPW // CR
CHT = 512


def _router_body(x_ref, wg_ref, r0f_ref, r1f_ref, r0b_ref, r1b_ref,
                 w1o_ref, w2o_ref, cnt_ref, loss_ref, big_ref):
    x = x_ref[...]
    logits = lax.dot_general(x, wg_ref[...], (((1,), (1,)), ((), ())))
    col = lax.broadcasted_iota(jnp.int32, (T, E), 1)
    m = jnp.max(logits, axis=1, keepdims=True)
    ex = jnp.exp(logits - m)
    probs = ex / jnp.sum(ex, axis=1, keepdims=True)
    p1 = jnp.max(probs, axis=1, keepdims=True)
    i1 = jnp.min(jnp.where(probs == p1, col, jnp.int32(E)), axis=1,
                 keepdims=True)
    probs2 = jnp.where(col == i1, -1.0, probs)
    p2 = jnp.max(probs2, axis=1, keepdims=True)
    i2 = jnp.min(jnp.where(probs2 == p2, col, jnp.int32(E)), axis=1,
                 keepdims=True)
    den = p1 + p2 + 1e-8
    w1 = p1 / den
    w2 = p2 / den
    oh1 = (col == i1).astype(jnp.float32)
    oh2 = (col == i2).astype(jnp.float32)
    occ = oh1 + oh2
    ri = lax.broadcasted_iota(jnp.int32, (CHT, CHT), 0)
    ci = lax.broadcasted_iota(jnp.int32, (CHT, CHT), 1)
    ltri = (ri > ci).astype(jnp.float32)
    carry = jnp.zeros((1, E), jnp.float32)
    chunks = []
    for ch in range(T // CHT):
        blk = lax.slice(occ, (ch * CHT, 0), ((ch + 1) * CHT, E))
        cum = lax.dot_general(ltri, blk, (((1,), (0,)), ((), ())),
                              preferred_element_type=jnp.float32) + carry
        chunks.append(cum)
        carry = carry + jnp.sum(blk, axis=0, keepdims=True)
    pos = jnp.concatenate(chunks, axis=0)
    counts = carry
    valsf = col.astype(jnp.float32) * jnp.float32(CAPF) + pos
    valsb = col.astype(jnp.float32) * jnp.float32(CAP) + pos
    r0f_ref[...] = jnp.sum(oh1 * valsf, axis=1, keepdims=True).astype(jnp.int32)
    r1f_ref[...] = jnp.sum(oh2 * valsf, axis=1, keepdims=True).astype(jnp.int32)
    r0b_ref[...] = jnp.sum(oh1 * valsb, axis=1, keepdims=True).astype(jnp.int32)
    r1b_ref[...] = jnp.sum(oh2 * valsb, axis=1, keepdims=True).astype(jnp.int32)
    w1o_ref[...] = jnp.broadcast_to(w1, (T, LN))
    w2o_ref[...] = jnp.broadcast_to(w2, (T, LN))
    cnt_ref[...] = counts.astype(jnp.int32)
    meanp = jnp.sum(probs, axis=0, keepdims=True) * jnp.float32(1.0 / T)
    usage = counts * jnp.float32(1.0 / (2 * T))
    loss_ref[...] = jnp.sum(meanp * usage, axis=1,
                            keepdims=True) * jnp.float32(E)
    big_ref[...] = jnp.max(counts, axis=1,
                           keepdims=True).astype(jnp.int32)


def _router_call(xf, wg):
    return pl.pallas_call(
        _router_body,
        out_shape=(
            jax.ShapeDtypeStruct((T, 1), jnp.int32),
            jax.ShapeDtypeStruct((T, 1), jnp.int32),
            jax.ShapeDtypeStruct((T, 1), jnp.int32),
            jax.ShapeDtypeStruct((T, 1), jnp.int32),
            jax.ShapeDtypeStruct((T, LN), jnp.float32),
            jax.ShapeDtypeStruct((T, LN), jnp.float32),
            jax.ShapeDtypeStruct((1, E), jnp.int32),
            jax.ShapeDtypeStruct((1, 1), jnp.float32),
            jax.ShapeDtypeStruct((1, 1), jnp.int32),
        ),
    )(xf, wg)


def _gelu(h):
    return 0.5 * h * (1.0 + lax.erf(h * 0.7071067811865476))


def _mlp_body(cnt_ref, xg_ref, wgt_ref, w1_ref, w2_ref, y_ref, *, nsub):
    e = pl.program_id(0)
    f = pl.program_id(1)
    c = cnt_ref[0, e]
    w1b = w1_ref[0].astype(jnp.bfloat16)
    w2b = w2_ref[0].astype(jnp.bfloat16)
    for sub in range(nsub):
        @pl.when(c > sub * SUB)
        def _():
            sl = pl.ds(sub * SUB, SUB)
            xs = xg_ref[sl, :].astype(jnp.bfloat16)
            hbs = []
            for hc in range(BF // HC):
                w1c = lax.slice(w1b, (hc * HC, 0), ((hc + 1) * HC, D))
                hcv = lax.dot_general(xs, w1c, (((1,), (1,)), ((), ())),
                                      preferred_element_type=jnp.float32)
                hbs.append(_gelu(hcv).astype(jnp.bfloat16))
            hb = jnp.concatenate(hbs, axis=1)
            part = lax.dot_general(hb, w2b, (((1,), (1,)), ((), ())),
                                   preferred_element_type=jnp.float32)

            @pl.when(f == 0)
            def _():
                y_ref[sl, :] = part

            @pl.when(jnp.logical_and(f > 0, f < KF - 1))
            def _():
                y_ref[sl, :] = y_ref[sl, :] + part

            @pl.when(f == KF - 1)
            def _():
                y_ref[sl, :] = (y_ref[sl, :] + part) * wgt_ref[sl, 0:1]


def _mlp_call(counts, xg, wgt, w1, w2, cap):
    grid_spec = pltpu.PrefetchScalarGridSpec(
        num_scalar_prefetch=1,
        grid=(E, KF),
        in_specs=[
            pl.BlockSpec((cap, D), lambda e, f, cnt: (e, 0)),
            pl.BlockSpec((cap, LN), lambda e, f, cnt: (e, 0)),
            pl.BlockSpec((1, BF, D), lambda e, f, cnt: (e, f, 0)),
            pl.BlockSpec((1, D, BF), lambda e, f, cnt: (e, 0, f)),
        ],
        out_specs=pl.BlockSpec((cap, D), lambda e, f, cnt: (e, 0)),
    )
    return pl.pallas_call(
        functools.partial(_mlp_body, nsub=cap // SUB),
        grid_spec=grid_spec,
        out_shape=jax.ShapeDtypeStruct((E * cap, D), jnp.float32),
        compiler_params=pltpu.CompilerParams(
            dimension_semantics=("arbitrary", "arbitrary")),
    )(counts, xg, wgt, w1, w2)


def _sc_dispatch_body(x_hbm, r0_hbm, r1_hbm, wa_hbm, wb_hbm, xg_hbm, wgt_hbm,
                      xbuf, wabuf, wbbuf, i0, i1, sem):
    wid = lax.axis_index("s") * 2 + lax.axis_index("c")
    base = wid * TPW
    pltpu.sync_copy(r0_hbm.at[pl.ds(base, TPW)], i0)
    pltpu.sync_copy(r1_hbm.at[pl.ds(base, TPW)], i1)
    pltpu.sync_copy(x_hbm.at[pl.ds(base, TPW)], xbuf)
    pltpu.sync_copy(wa_hbm.at[pl.ds(base, TPW)], wabuf)
    pltpu.sync_copy(wb_hbm.at[pl.ds(base, TPW)], wbbuf)
    c1 = pltpu.async_copy(xbuf, xg_hbm.at[i0], sem)
    c2 = pltpu.async_copy(xbuf, xg_hbm.at[i1], sem)
    c3 = pltpu.async_copy(wabuf, wgt_hbm.at[i0], sem)
    c4 = pltpu.async_copy(wbbuf, wgt_hbm.at[i1], sem)
    c1.wait()
    c2.wait()
    c3.wait()
    c4.wait()


def _sc_dispatch(xf, r0f, r1f, w1b, w2b, cap):
    mesh = plsc.VectorSubcoreMesh(core_axis_name="c", subcore_axis_name="s")
    fn = functools.partial(
        pl.kernel,
        mesh=mesh,
        out_type=(
            jax.ShapeDtypeStruct((E * cap, D), jnp.float32),
            jax.ShapeDtypeStruct((E * cap, LN), jnp.float32),
        ),
        scratch_types=[
            pltpu.VMEM((TPW, D), jnp.float32),
            pltpu.VMEM((TPW, LN), jnp.float32),
            pltpu.VMEM((TPW, LN), jnp.float32),
            pltpu.VMEM((TPW,), jnp.int32),
            pltpu.VMEM((TPW,), jnp.int32),
            pltpu.SemaphoreType.DMA,
        ],
    )(_sc_dispatch_body)
    return fn(xf, r0f, r1f, w1b, w2b)


def _sc_combine_body(y_hbm, r0_hbm, r1_hbm, o_hbm, b0, b1, i0, i1, sem0,
                     sem1):
    wid = lax.axis_index("s") * 2 + lax.axis_index("c")
    base = wid * TPW
    pltpu.sync_copy(r0_hbm.at[pl.ds(base, TPW)], i0)
    pltpu.sync_copy(r1_hbm.at[pl.ds(base, TPW)], i1)
    cps = [None] * CQ

    def _fire(q):
        csl = pl.ds(q * CR, CR)
        rb = q % 2
        return (pltpu.async_copy(y_hbm.at[i0.at[csl]], b0.at[rb], sem0),
                pltpu.async_copy(y_hbm.at[i1.at[csl]], b1.at[rb], sem1))

    cps[0] = _fire(0)
    cps[1] = _fire(1)
    for q in range(CQ):
        c0, c1 = cps[q]
        c0.wait()
        c1.wait()
        rb = q % 2

        def body(r, _):
            for k in range(D // 16):
                ksl = pl.ds(k * 16, 16)
                b0[rb, r, ksl] = b0[rb, r, ksl] + b1[rb, r, ksl]
            return 0

        lax.fori_loop(0, CR, body, 0)
        pltpu.sync_copy(b0.at[rb], o_hbm.at[pl.ds(base + q * CR, CR)])
        if q + 2 < CQ:
            cps[q + 2] = _fire(q + 2)


def _sc_combine(y, r0f, r1f):
    mesh = plsc.VectorSubcoreMesh(core_axis_name="c", subcore_axis_name="s")
    fn = functools.partial(
        pl.kernel,
        mesh=mesh,
        out_type=jax.ShapeDtypeStruct((T, D), jnp.float32),
        scratch_types=[
            pltpu.VMEM((2, CR, D), jnp.float32),
            pltpu.VMEM((2, CR, D), jnp.float32),
            pltpu.VMEM((TPW,), jnp.int32),
            pltpu.VMEM((TPW,), jnp.int32),
            pltpu.SemaphoreType.DMA,
            pltpu.SemaphoreType.DMA,
        ],
    )(_sc_combine_body)
    return fn(y, r0f, r1f)


def _moe_branch(cap):
    def run(xf, r0, r1, w1b, w2b, counts, W1, W2):
        r0f = r0.reshape(T)
        r1f = r1.reshape(T)
        xg, wgt = _sc_dispatch(xf, r0f, r1f, w1b, w2b, cap)
        y = _mlp_call(counts, xg, wgt, W1, W2, cap)
        return _sc_combine(y, r0f, r1f)
    return run


def kernel(x, Wg, W1, W2):
    B, S, _ = x.shape
    xf = x.reshape(T, D)
    (r0f_, r1f_, r0b_, r1b_, w1b, w2b, counts, loss,
     maxc) = _router_call(xf, Wg)
    out = lax.cond(
        maxc[0, 0] > CAPF,
        lambda: _moe_branch(CAP)(xf, r0b_, r1b_, w1b, w2b, counts, W1, W2),
        lambda: _moe_branch(CAPF)(xf, r0f_, r1f_, w1b, w2b, counts, W1, W2),
    )
    return out.reshape(B, S, D), loss[0, 0]


# speedup vs baseline: 2.0437x; 1.0944x over previous
import functools

import jax
import jax.numpy as jnp
from jax import lax
from jax.experimental import pallas as pl
from jax.experimental.pallas import tpu as pltpu
from jax.experimental.pallas import tpu_sc as plsc

T = 2048
D = 1024
F = 4096
E = 8
LN = 128
CAP = 2048
CAPF = 768
BFF = 2048
BFB = 1024
HC = 256
SUB = 256
NW = 32
TPW = T // NW
CR = 16
CQ = TPW // CR
CHT = 512


def _router_body(x_ref, wg_ref, r0f_ref, r1f_ref, r0b_ref, r1b_ref,
                 w1o_ref, w2o_ref, cnt_ref, loss_ref, big_ref):
    x = x_ref[...]
    logits = lax.dot_general(x, wg_ref[...], (((1,), (1,)), ((), ())))
    col = lax.broadcasted_iota(jnp.int32, (T, E), 1)
    m = jnp.max(logits, axis=1, keepdims=True)
    ex = jnp.exp(logits - m)
    probs = ex / jnp.sum(ex, axis=1, keepdims=True)
    p1 = jnp.max(probs, axis=1, keepdims=True)
    i1 = jnp.min(jnp.where(probs == p1, col, jnp.int32(E)), axis=1,
                 keepdims=True)
    probs2 = jnp.where(col == i1, -1.0, probs)
    p2 = jnp.max(probs2, axis=1, keepdims=True)
    i2 = jnp.min(jnp.where(probs2 == p2, col, jnp.int32(E)), axis=1,
                 keepdims=True)
    den = p1 + p2 + 1e-8
    w1 = p1 / den
    w2 = p2 / den
    oh1 = (col == i1).astype(jnp.float32)
    oh2 = (col == i2).astype(jnp.float32)
    occ = oh1 + oh2
    ri = lax.broadcasted_iota(jnp.int32, (CHT, CHT), 0)
    ci = lax.broadcasted_iota(jnp.int32, (CHT, CHT), 1)
    ltri = (ri > ci).astype(jnp.float32)
    carry = jnp.zeros((1, E), jnp.float32)
    chunks = []
    for ch in range(T // CHT):
        blk = lax.slice(occ, (ch * CHT, 0), ((ch + 1) * CHT, E))
        cum = lax.dot_general(ltri, blk, (((1,), (0,)), ((), ())),
                              preferred_element_type=jnp.float32) + carry
        chunks.append(cum)
        carry = carry + jnp.sum(blk, axis=0, keepdims=True)
    pos = jnp.concatenate(chunks, axis=0)
    counts = carry
    valsf = col.astype(jnp.float32) * jnp.float32(CAPF) + pos
    valsb = col.astype(jnp.float32) * jnp.float32(CAP) + pos
    r0f_ref[...] = jnp.sum(oh1 * valsf, axis=1, keepdims=True).astype(jnp.int32)
    r1f_ref[...] = jnp.sum(oh2 * valsf, axis=1, keepdims=True).astype(jnp.int32)
    r0b_ref[...] = jnp.sum(oh1 * valsb, axis=1, keepdims=True).astype(jnp.int32)
    r1b_ref[...] = jnp.sum(oh2 * valsb, axis=1, keepdims=True).astype(jnp.int32)
    w1o_ref[...] = jnp.broadcast_to(w1, (T, LN))
    w2o_ref[...] = jnp.broadcast_to(w2, (T, LN))
    cnt_ref[...] = counts.astype(jnp.int32)
    meanp = jnp.sum(probs, axis=0, keepdims=True) * jnp.float32(1.0 / T)
    usage = counts * jnp.float32(1.0 / (2 * T))
    loss_ref[...] = jnp.sum(meanp * usage, axis=1,
                            keepdims=True) * jnp.float32(E)
    big_ref[...] = jnp.max(counts, axis=1,
                           keepdims=True).astype(jnp.int32)


def _router_call(xf, wg):
    return pl.pallas_call(
        _router_body,
        out_shape=(
            jax.ShapeDtypeStruct((T, 1), jnp.int32),
            jax.ShapeDtypeStruct((T, 1), jnp.int32),
            jax.ShapeDtypeStruct((T, 1), jnp.int32),
            jax.ShapeDtypeStruct((T, 1), jnp.int32),
            jax.ShapeDtypeStruct((T, LN), jnp.float32),
            jax.ShapeDtypeStruct((T, LN), jnp.float32),
            jax.ShapeDtypeStruct((1, E), jnp.int32),
            jax.ShapeDtypeStruct((1, 1), jnp.float32),
            jax.ShapeDtypeStruct((1, 1), jnp.int32),
        ),
    )(xf, wg)


def _gelu(h):
    return 0.5 * h * (1.0 + lax.erf(h * 0.7071067811865476))


def _mlp_body(cnt_ref, xg_ref, wgt_ref, w1_ref, w2_ref, y_ref, *, nsub, bf):
    e = pl.program_id(0)
    f = pl.program_id(1)
    kf = F // bf
    c = cnt_ref[0, e]
    w1b = w1_ref[0].astype(jnp.bfloat16)
    w2b = w2_ref[0].astype(jnp.bfloat16)
    for sub in range(nsub):
        @pl.when(c > sub * SUB)
        def _():
            sl = pl.ds(sub * SUB, SUB)
            xs = xg_ref[sl, :].astype(jnp.bfloat16)
            hbs = []
            for hc in range(bf // HC):
                w1c = lax.slice(w1b, (hc * HC, 0), ((hc + 1) * HC, D))
                hcv = lax.dot_general(xs, w1c, (((1,), (1,)), ((), ())),
                                      preferred_element_type=jnp.float32)
                hbs.append(_gelu(hcv).astype(jnp.bfloat16))
            hb = jnp.concatenate(hbs, axis=1)
            part = lax.dot_general(hb, w2b, (((1,), (1,)), ((), ())),
                                   preferred_element_type=jnp.float32)

            @pl.when(f == 0)
            def _():
                y_ref[sl, :] = part

            @pl.when(jnp.logical_and(f > 0, f < kf - 1))
            def _():
                y_ref[sl, :] = y_ref[sl, :] + part

            @pl.when(f == kf - 1)
            def _():
                y_ref[sl, :] = (y_ref[sl, :] + part) * wgt_ref[sl, 0:1]


def _mlp_call(counts, xg, wgt, w1, w2, cap, bf):
    grid_spec = pltpu.PrefetchScalarGridSpec(
        num_scalar_prefetch=1,
        grid=(E, F // bf),
        in_specs=[
            pl.BlockSpec((cap, D), lambda e, f, cnt: (e, 0)),
            pl.BlockSpec((cap, LN), lambda e, f, cnt: (e, 0)),
            pl.BlockSpec((1, bf, D), lambda e, f, cnt: (e, f, 0)),
            pl.BlockSpec((1, D, bf), lambda e, f, cnt: (e, 0, f)),
        ],
        out_specs=pl.BlockSpec((cap, D), lambda e, f, cnt: (e, 0)),
    )
    return pl.pallas_call(
        functools.partial(_mlp_body, nsub=cap // SUB, bf=bf),
        grid_spec=grid_spec,
        out_shape=jax.ShapeDtypeStruct((E * cap, D), jnp.float32),
        compiler_params=pltpu.CompilerParams(
            dimension_semantics=("arbitrary", "arbitrary")),
    )(counts, xg, wgt, w1, w2)


def _sc_dispatch_body(x_hbm, r0_hbm, r1_hbm, wa_hbm, wb_hbm, xg_hbm, wgt_hbm,
                      xbuf, wabuf, wbbuf, i0, i1, sem):
    wid = lax.axis_index("s") * 2 + lax.axis_index("c")
    base = wid * TPW
    pltpu.sync_copy(r0_hbm.at[pl.ds(base, TPW)], i0)
    pltpu.sync_copy(r1_hbm.at[pl.ds(base, TPW)], i1)
    pltpu.sync_copy(x_hbm.at[pl.ds(base, TPW)], xbuf)
    pltpu.sync_copy(wa_hbm.at[pl.ds(base, TPW)], wabuf)
    pltpu.sync_copy(wb_hbm.at[pl.ds(base, TPW)], wbbuf)
    c1 = pltpu.async_copy(xbuf, xg_hbm.at[i0], sem)
    c2 = pltpu.async_copy(xbuf, xg_hbm.at[i1], sem)
    c3 = pltpu.async_copy(wabuf, wgt_hbm.at[i0], sem)
    c4 = pltpu.async_copy(wbbuf, wgt_hbm.at[i1], sem)
    c1.wait()
    c2.wait()
    c3.wait()
    c4.wait()


def _sc_dispatch(xf, r0f, r1f, w1b, w2b, cap):
    mesh = plsc.VectorSubcoreMesh(core_axis_name="c", subcore_axis_name="s")
    fn = functools.partial(
        pl.kernel,
        mesh=mesh,
        out_type=(
            jax.ShapeDtypeStruct((E * cap, D), jnp.float32),
            jax.ShapeDtypeStruct((E * cap, LN), jnp.float32),
        ),
        scratch_types=[
            pltpu.VMEM((TPW, D), jnp.float32),
            pltpu.VMEM((TPW, LN), jnp.float32),
            pltpu.VMEM((TPW, LN), jnp.float32),
            pltpu.VMEM((TPW,), jnp.int32),
            pltpu.VMEM((TPW,), jnp.int32),
            pltpu.SemaphoreType.DMA,
        ],
    )(_sc_dispatch_body)
    return fn(xf, r0f, r1f, w1b, w2b)


def _sc_combine_body(y_hbm, r0_hbm, r1_hbm, o_hbm, b0, b1, i0, i1, sem0,
                     sem1):
    wid = lax.axis_index("s") * 2 + lax.axis_index("c")
    base = wid * TPW
    pltpu.sync_copy(r0_hbm.at[pl.ds(base, TPW)], i0)
    pltpu.sync_copy(r1_hbm.at[pl.ds(base, TPW)], i1)
    cps = [None] * CQ

    def _fire(q):
        csl = pl.ds(q * CR, CR)
        rb = q % 2
        return (pltpu.async_copy(y_hbm.at[i0.at[csl]], b0.at[rb], sem0),
                pltpu.async_copy(y_hbm.at[i1.at[csl]], b1.at[rb], sem1))

    cps[0] = _fire(0)
    cps[1] = _fire(1)
    for q in range(CQ):
        c0, c1 = cps[q]
        c0.wait()
        c1.wait()
        rb = q % 2

        def body(r, _):
            for k in range(D // 16):
                ksl = pl.ds(k * 16, 16)
                b0[rb, r, ksl] = b0[rb, r, ksl] + b1[rb, r, ksl]
            return 0

        lax.fori_loop(0, CR, body, 0)
        pltpu.sync_copy(b0.at[rb], o_hbm.at[pl.ds(base + q * CR, CR)])
        if q + 2 < CQ:
            cps[q + 2] = _fire(q + 2)


def _sc_combine(y, r0f, r1f):
    mesh = plsc.VectorSubcoreMesh(core_axis_name="c", subcore_axis_name="s")
    fn = functools.partial(
        pl.kernel,
        mesh=mesh,
        out_type=jax.ShapeDtypeStruct((T, D), jnp.float32),
        scratch_types=[
            pltpu.VMEM((2, CR, D), jnp.float32),
            pltpu.VMEM((2, CR, D), jnp.float32),
            pltpu.VMEM((TPW,), jnp.int32),
            pltpu.VMEM((TPW,), jnp.int32),
            pltpu.SemaphoreType.DMA,
            pltpu.SemaphoreType.DMA,
        ],
    )(_sc_combine_body)
    return fn(y, r0f, r1f)


def _moe_branch(cap, bf):
    def run(xf, r0, r1, w1b, w2b, counts, W1, W2):
        r0f = r0.reshape(T)
        r1f = r1.reshape(T)
        xg, wgt = _sc_dispatch(xf, r0f, r1f, w1b, w2b, cap)
        y = _mlp_call(counts, xg, wgt, W1, W2, cap, bf)
        return _sc_combine(y, r0f, r1f)
    return run


def kernel(x, Wg, W1, W2):
    B, S, _ = x.shape
    xf = x.reshape(T, D)
    (r0f_, r1f_, r0b_, r1b_, w1b, w2b, counts, loss,
     maxc) = _router_call(xf, Wg)
    out = lax.cond(
        maxc[0, 0] > CAPF,
        lambda: _moe_branch(CAP, BFB)(xf, r0b_, r1b_, w1b, w2b, counts,
                                      W1, W2),
        lambda: _moe_branch(CAPF, BFF)(xf, r0f_, r1f_, w1b, w2b, counts,
                                       W1, W2),
    )
    return out.reshape(B, S, D), loss[0, 0]
